# Initial kernel scaffold; baseline (speedup 1.0000x reference)
import jax
import jax.numpy as jnp
from jax.experimental import pallas as pl


def kernel(rgb_samples, radiance_samples, ray_samples_z, ray_samples_dt, ray_t_exit, segment_ids, use_ray_t_exit):
    raise NotImplementedError("write your pallas kernel here")


# speedup vs baseline: 11.6748x; 11.6748x over previous
import functools

import jax
import jax.numpy as jnp
from jax import lax
from jax.experimental import pallas as pl
from jax.experimental.pallas import tpu as pltpu
from jax.experimental.pallas import tpu_sc as plsc

N_RAYS = 4096
TOTAL = 262144
NC = 2
NS = 16
NW = NC * NS
L = 16
CHUNK = TOTAL // NW
NV = CHUNK // L
RPT = N_RAYS // NW
F32 = jnp.float32
I32 = jnp.int32


def _mesh():
    return plsc.VectorSubcoreMesh(
        core_axis_name="c", subcore_axis_name="s",
        num_cores=NC, num_subcores=NS)


def _wid():
    return lax.axis_index("s") * NC + lax.axis_index("c")


@functools.partial(
    pl.kernel,
    out_type=jax.ShapeDtypeStruct((NW, N_RAYS), F32),
    mesh=_mesh(),
    compiler_params=pltpu.CompilerParams(needs_layout_passes=False),
    scratch_types=[
        pltpu.VMEM((CHUNK,), F32),
        pltpu.VMEM((CHUNK,), F32),
        pltpu.VMEM((CHUNK,), I32),
        pltpu.VMEM((N_RAYS,), F32),
    ],
)
def _k1(sigma_h, dt_h, seg_h, part_h, sig_v, dt_v, seg_v, acc_v):
    w = _wid()
    base = w * CHUNK
    pltpu.sync_copy(sigma_h.at[pl.ds(base, CHUNK)], sig_v)
    pltpu.sync_copy(dt_h.at[pl.ds(base, CHUNK)], dt_v)
    pltpu.sync_copy(seg_h.at[pl.ds(base, CHUNK)], seg_v)

    zv = jnp.zeros((L,), F32)

    def zbody(i, c):
        acc_v[pl.ds(i * L, L)] = zv
        return c

    lax.fori_loop(0, N_RAYS // L, zbody, 0)

    def body(i, c):
        sl = pl.ds(i * L, L)
        s = sig_v[sl] * dt_v[sl]
        plsc.addupdate_scatter(acc_v, [seg_v[sl]], s)
        return c

    lax.fori_loop(0, NV, body, 0)
    pltpu.sync_copy(acc_v, part_h.at[w])


@functools.partial(
    pl.kernel,
    out_type=(jax.ShapeDtypeStruct((TOTAL,), F32),
              jax.ShapeDtypeStruct((NW, N_RAYS * 4), F32)),
    mesh=_mesh(),
    compiler_params=pltpu.CompilerParams(needs_layout_passes=False),
    scratch_types=[
        pltpu.VMEM((CHUNK,), F32),
        pltpu.VMEM((CHUNK,), F32),
        pltpu.VMEM((CHUNK + L,), I32),
        pltpu.VMEM((CHUNK,), F32),
        pltpu.VMEM((CHUNK * 3,), F32),
        pltpu.VMEM((CHUNK,), F32),
        pltpu.VMEM((CHUNK,), F32),
        pltpu.VMEM((CHUNK,), F32),
        pltpu.VMEM((N_RAYS,), F32),
        pltpu.VMEM((N_RAYS * 4,), F32),
        pltpu.VMEM((NW,), I32),
        pltpu.VMEM((NW,), F32),
        pltpu.SemaphoreType.DMA,
    ],
)
def _k2(sigma_h, dt_h, seg_h, z_h, rgb_h, partf_h, wout_h, part4_h,
        sig_v, dt_v, segb_v, z_v, rgb_v, s_v, ex_v, w_v, start_v, acc4_v,
        idx_v, col_v, sem):
    w = _wid()
    base = w * CHUNK
    pltpu.sync_copy(sigma_h.at[pl.ds(base, CHUNK)], sig_v)
    pltpu.sync_copy(dt_h.at[pl.ds(base, CHUNK)], dt_v)
    pltpu.sync_copy(seg_h.at[pl.ds(base, CHUNK)], segb_v.at[pl.ds(L, CHUNK)])
    pltpu.sync_copy(z_h.at[pl.ds(base, CHUNK)], z_v)
    pltpu.sync_copy(rgb_h.at[pl.ds(base * 3, CHUNK * 3)], rgb_v)
    segb_v[pl.ds(0, L)] = jnp.full((L,), -1, I32)

    zv = jnp.zeros((L,), F32)

    def zbody(i, c):
        acc4_v[pl.ds(i * L, L)] = zv
        return c

    lax.fori_loop(0, N_RAYS * 4 // L, zbody, 0)

    def scanb(i, carry):
        sl = pl.ds(i * L, L)
        s = sig_v[sl] * dt_v[sl]
        s_v[sl] = s
        inc = plsc.cumsum(s)
        ex = inc - s + carry
        ex_v[sl] = ex
        seg = segb_v[pl.ds(i * L + L, L)]
        segp = segb_v[pl.ds(i * L + L - 1, L)]
        plsc.store_scatter(start_v, [seg], ex, mask=seg != segp)
        return carry + jnp.sum(s)

    lax.fori_loop(0, NV, scanb, F32(0.0))

    seg0 = segb_v[pl.ds(L, L)]
    r0 = jnp.min(seg0)
    lanes = lax.iota(I32, L)
    idx_v[pl.ds(0, L)] = lanes * N_RAYS + r0
    idx_v[pl.ds(L, L)] = (lanes + L) * N_RAYS + r0
    pltpu.async_copy(partf_h.at[idx_v], col_v, sem).wait()
    c0 = col_v[pl.ds(0, L)]
    c1 = col_v[pl.ds(L, L)]
    cin = (jnp.sum(jnp.where(lanes < w, c0, 0.0)) +
           jnp.sum(jnp.where(lanes + L < w, c1, 0.0)))
    plsc.store_scatter(start_v, [seg0], jnp.full((L,), -cin, F32),
                       mask=lanes == 0)

    def wbody(i, c):
        sl = pl.ds(i * L, L)
        s = s_v[sl]
        seg = segb_v[pl.ds(i * L + L, L)]
        sa = plsc.load_gather(start_v, [seg])
        trans = jnp.exp(sa - ex_v[sl])
        alpha = 1.0 - jnp.exp(-s)
        wgt = trans * alpha
        w_v[sl] = wgt
        gi = (lanes + i * L) * 3
        r = plsc.load_gather(rgb_v, [gi])
        g = plsc.load_gather(rgb_v, [gi + 1])
        b = plsc.load_gather(rgb_v, [gi + 2])
        s4 = seg * 4
        plsc.addupdate_scatter(acc4_v, [s4], wgt * r)
        plsc.addupdate_scatter(acc4_v, [s4 + 1], wgt * g)
        plsc.addupdate_scatter(acc4_v, [s4 + 2], wgt * b)
        plsc.addupdate_scatter(acc4_v, [s4 + 3], wgt * z_v[sl])
        return c

    lax.fori_loop(0, NV, wbody, 0)
    pltpu.sync_copy(w_v, wout_h.at[pl.ds(base, CHUNK)])
    pltpu.sync_copy(acc4_v, part4_h.at[w])


@functools.partial(
    pl.kernel,
    out_type=(jax.ShapeDtypeStruct((N_RAYS * 3,), F32),
              jax.ShapeDtypeStruct((N_RAYS,), F32),
              jax.ShapeDtypeStruct((N_RAYS,), F32)),
    mesh=_mesh(),
    compiler_params=pltpu.CompilerParams(needs_layout_passes=False),
    scratch_types=[
        pltpu.VMEM((NW * RPT,), F32),
        pltpu.VMEM((NW * RPT * 4,), F32),
        pltpu.VMEM((RPT,), F32),
        pltpu.VMEM((RPT * 4,), F32),
        pltpu.VMEM((RPT,), F32),
        pltpu.VMEM((L,), I32),
        pltpu.VMEM((RPT * 3,), F32),
        pltpu.VMEM((RPT,), F32),
        pltpu.VMEM((RPT,), F32),
    ],
)
def _k3(part_h, part4_h, texit_h, use_h, rgb_o, dep_o, bg_o,
        ps_v, pc_v, tot_v, totc_v, tex_v, use_v, rgbb_v, depb_v, bgb_v):
    w = _wid()
    rbase = w * RPT
    for tp in range(NW):
        pltpu.sync_copy(part_h.at[tp, pl.ds(rbase, RPT)],
                        ps_v.at[pl.ds(tp * RPT, RPT)])
        pltpu.sync_copy(part4_h.at[tp, pl.ds(rbase * 4, RPT * 4)],
                        pc_v.at[pl.ds(tp * RPT * 4, RPT * 4)])
    pltpu.sync_copy(texit_h.at[pl.ds(rbase, RPT)], tex_v)
    pltpu.sync_copy(use_h, use_v)

    zv = jnp.zeros((L,), F32)

    def sum_s(j, c):
        def inner(tp, acc):
            return acc + ps_v[pl.ds(tp * RPT + j * L, L)]
        tot_v[pl.ds(j * L, L)] = lax.fori_loop(0, NW, inner, zv)
        return c

    lax.fori_loop(0, RPT // L, sum_s, 0)

    def sum_c(j, c):
        def inner(tp, acc):
            return acc + pc_v[pl.ds(tp * RPT * 4 + j * L, L)]
        totc_v[pl.ds(j * L, L)] = lax.fori_loop(0, NW, inner, zv)
        return c

    lax.fori_loop(0, RPT * 4 // L, sum_c, 0)

    lanes = lax.iota(I32, L)
    u = use_v[pl.ds(0, L)]

    def fin(j, c):
        sl = pl.ds(j * L, L)
        bg = jnp.exp(-tot_v[sl])
        gi = (lanes + j * L) * 4
        wr = plsc.load_gather(totc_v, [gi])
        wg = plsc.load_gather(totc_v, [gi + 1])
        wb = plsc.load_gather(totc_v, [gi + 2])
        wz = plsc.load_gather(totc_v, [gi + 3])
        dep = jnp.where(u != 0, wz + bg * tex_v[sl], wz)
        bgb_v[sl] = bg
        depb_v[sl] = dep
        oi = (lanes + j * L) * 3
        plsc.store_scatter(rgbb_v, [oi], wr)
        plsc.store_scatter(rgbb_v, [oi + 1], wg)
        plsc.store_scatter(rgbb_v, [oi + 2], wb)
        return c

    lax.fori_loop(0, RPT // L, fin, 0)
    pltpu.sync_copy(rgbb_v, rgb_o.at[pl.ds(rbase * 3, RPT * 3)])
    pltpu.sync_copy(depb_v, dep_o.at[pl.ds(rbase, RPT)])
    pltpu.sync_copy(bgb_v, bg_o.at[pl.ds(rbase, RPT)])


def kernel(rgb_samples, radiance_samples, ray_samples_z, ray_samples_dt,
           ray_t_exit, segment_ids, use_ray_t_exit):
    sigma = radiance_samples.reshape(TOTAL)
    rgbf = rgb_samples.reshape(TOTAL * 3)
    part = _k1(sigma, ray_samples_dt, segment_ids)
    partf = part.reshape(NW * N_RAYS)
    wout, part4 = _k2(sigma, ray_samples_dt, segment_ids, ray_samples_z,
                      rgbf, partf)
    use_arr = jnp.full((L,), use_ray_t_exit, I32)
    rgbo, dep, bg = _k3(part, part4, ray_t_exit.reshape(N_RAYS), use_arr)
    return (rgbo.reshape(N_RAYS, 3), dep.reshape(N_RAYS, 1),
            bg.reshape(N_RAYS, 1), wout.reshape(TOTAL, 1))


# speedup vs baseline: 26.7726x; 2.2932x over previous
import functools

import jax
import jax.numpy as jnp
from jax import lax
from jax.experimental import pallas as pl
from jax.experimental.pallas import tpu as pltpu
from jax.experimental.pallas import tpu_sc as plsc

N_RAYS = 4096
TOTAL = 262144
NC = 2
NS = 16
NW = NC * NS
L = 16
CHUNK = TOTAL // NW
NV = CHUNK // L
RPT = N_RAYS // NW
F32 = jnp.float32
I32 = jnp.int32


def _mesh():
    return plsc.VectorSubcoreMesh(
        core_axis_name="c", subcore_axis_name="s",
        num_cores=NC, num_subcores=NS)


def _wid():
    return lax.axis_index("s") * NC + lax.axis_index("c")


@functools.partial(
    pl.kernel,
    out_type=jax.ShapeDtypeStruct((NW, N_RAYS), F32),
    mesh=_mesh(),
    compiler_params=pltpu.CompilerParams(needs_layout_passes=False),
    scratch_types=[
        pltpu.VMEM((CHUNK,), F32),
        pltpu.VMEM((CHUNK,), F32),
        pltpu.VMEM((CHUNK,), I32),
        pltpu.VMEM((N_RAYS,), F32),
    ],
)
def _k1(sigma_h, dt_h, seg_h, part_h, sig_v, dt_v, seg_v, acc_v):
    w = _wid()
    base = w * CHUNK
    pltpu.sync_copy(sigma_h.at[pl.ds(base, CHUNK)], sig_v)
    pltpu.sync_copy(dt_h.at[pl.ds(base, CHUNK)], dt_v)
    pltpu.sync_copy(seg_h.at[pl.ds(base, CHUNK)], seg_v)

    zv = jnp.zeros((L,), F32)

    @plsc.parallel_loop(0, N_RAYS // L, unroll=8)
    def _(i):
        acc_v[pl.ds(i * L, L)] = zv

    @plsc.parallel_loop(0, NV, unroll=4)
    def _(i):
        sl = pl.ds(i * L, L)
        s = sig_v[sl] * dt_v[sl]
        plsc.addupdate_scatter(acc_v, [seg_v[sl]], s)

    pltpu.sync_copy(acc_v, part_h.at[w])


@functools.partial(
    pl.kernel,
    out_type=(jax.ShapeDtypeStruct((TOTAL,), F32),
              jax.ShapeDtypeStruct((NW, N_RAYS * 4), F32)),
    mesh=_mesh(),
    compiler_params=pltpu.CompilerParams(needs_layout_passes=False),
    scratch_types=[
        pltpu.VMEM((CHUNK,), F32),
        pltpu.VMEM((CHUNK,), F32),
        pltpu.VMEM((CHUNK + L,), I32),
        pltpu.VMEM((CHUNK,), F32),
        pltpu.VMEM((CHUNK,), F32),
        pltpu.VMEM((CHUNK,), F32),
        pltpu.VMEM((CHUNK,), F32),
        pltpu.VMEM((CHUNK,), F32),
        pltpu.VMEM((CHUNK,), F32),
        pltpu.VMEM((CHUNK,), F32),
        pltpu.VMEM((N_RAYS,), F32),
        pltpu.VMEM((N_RAYS * 4,), F32),
        pltpu.VMEM((NW,), I32),
        pltpu.VMEM((NW,), F32),
        pltpu.SemaphoreType.DMA,
    ],
)
def _k2(sigma_h, dt_h, seg_h, z_h, rgb_h, partf_h, wout_h, part4_h,
        sig_v, dt_v, segb_v, z_v, r_v, g_v, b_v, s_v, ex_v, w_v, start_v,
        acc4_v, idx_v, col_v, sem):
    w = _wid()
    base = w * CHUNK
    pltpu.sync_copy(sigma_h.at[pl.ds(base, CHUNK)], sig_v)
    pltpu.sync_copy(dt_h.at[pl.ds(base, CHUNK)], dt_v)
    pltpu.sync_copy(seg_h.at[pl.ds(base, CHUNK)], segb_v.at[pl.ds(L, CHUNK)])
    pltpu.sync_copy(z_h.at[pl.ds(base, CHUNK)], z_v)
    pltpu.sync_copy(rgb_h.at[pl.ds(base, CHUNK)], r_v)
    pltpu.sync_copy(rgb_h.at[pl.ds(TOTAL + base, CHUNK)], g_v)
    pltpu.sync_copy(rgb_h.at[pl.ds(2 * TOTAL + base, CHUNK)], b_v)
    segb_v[pl.ds(0, L)] = jnp.full((L,), -1, I32)

    zv = jnp.zeros((L,), F32)

    @plsc.parallel_loop(0, N_RAYS * 4 // L, unroll=8)
    def _(i):
        acc4_v[pl.ds(i * L, L)] = zv

    @plsc.parallel_loop(0, NV, unroll=4, carry=F32(0.0))
    def _(i, carry):
        sl = pl.ds(i * L, L)
        s = sig_v[sl] * dt_v[sl]
        s_v[sl] = s
        inc = plsc.cumsum(s)
        ex = inc - s + carry
        ex_v[sl] = ex
        seg = segb_v[pl.ds(i * L + L, L)]
        segp = segb_v[pl.ds(i * L + L - 1, L)]
        plsc.store_scatter(start_v, [seg], ex, mask=seg != segp)
        return carry + jnp.sum(s)

    seg0 = segb_v[pl.ds(L, L)]
    r0 = jnp.min(seg0)
    lanes = lax.iota(I32, L)
    idx_v[pl.ds(0, L)] = lanes * N_RAYS + r0
    idx_v[pl.ds(L, L)] = (lanes + L) * N_RAYS + r0
    pltpu.async_copy(partf_h.at[idx_v], col_v, sem).wait()
    c0 = col_v[pl.ds(0, L)]
    c1 = col_v[pl.ds(L, L)]
    cin = (jnp.sum(jnp.where(lanes < w, c0, 0.0)) +
           jnp.sum(jnp.where(lanes + L < w, c1, 0.0)))
    plsc.store_scatter(start_v, [seg0], jnp.full((L,), -cin, F32),
                       mask=lanes == 0)

    @plsc.parallel_loop(0, NV, unroll=4)
    def _(i):
        sl = pl.ds(i * L, L)
        s = s_v[sl]
        seg = segb_v[pl.ds(i * L + L, L)]
        sa = plsc.load_gather(start_v, [seg])
        trans = jnp.exp(sa - ex_v[sl])
        alpha = 1.0 - jnp.exp(-s)
        wgt = trans * alpha
        w_v[sl] = wgt
        s4 = seg * 4
        plsc.addupdate_scatter(acc4_v, [s4], wgt * r_v[sl])
        plsc.addupdate_scatter(acc4_v, [s4 + 1], wgt * g_v[sl])
        plsc.addupdate_scatter(acc4_v, [s4 + 2], wgt * b_v[sl])
        plsc.addupdate_scatter(acc4_v, [s4 + 3], wgt * z_v[sl])

    pltpu.sync_copy(w_v, wout_h.at[pl.ds(base, CHUNK)])
    pltpu.sync_copy(acc4_v, part4_h.at[w])


@functools.partial(
    pl.kernel,
    out_type=(jax.ShapeDtypeStruct((N_RAYS * 3,), F32),
              jax.ShapeDtypeStruct((N_RAYS,), F32),
              jax.ShapeDtypeStruct((N_RAYS,), F32)),
    mesh=_mesh(),
    compiler_params=pltpu.CompilerParams(needs_layout_passes=False),
    scratch_types=[
        pltpu.VMEM((NW * RPT,), F32),
        pltpu.VMEM((NW * RPT * 4,), F32),
        pltpu.VMEM((RPT,), F32),
        pltpu.VMEM((RPT * 4,), F32),
        pltpu.VMEM((RPT,), F32),
        pltpu.VMEM((L,), I32),
        pltpu.VMEM((RPT * 3,), F32),
        pltpu.VMEM((RPT,), F32),
        pltpu.VMEM((RPT,), F32),
    ],
)
def _k3(part_h, part4_h, texit_h, use_h, rgb_o, dep_o, bg_o,
        ps_v, pc_v, tot_v, totc_v, tex_v, use_v, rgbb_v, depb_v, bgb_v):
    w = _wid()
    rbase = w * RPT
    for tp in range(NW):
        pltpu.sync_copy(part_h.at[tp, pl.ds(rbase, RPT)],
                        ps_v.at[pl.ds(tp * RPT, RPT)])
        pltpu.sync_copy(part4_h.at[tp, pl.ds(rbase * 4, RPT * 4)],
                        pc_v.at[pl.ds(tp * RPT * 4, RPT * 4)])
    pltpu.sync_copy(texit_h.at[pl.ds(rbase, RPT)], tex_v)
    pltpu.sync_copy(use_h, use_v)

    zv = jnp.zeros((L,), F32)

    @plsc.parallel_loop(0, RPT // L)
    def _(j):
        acc = zv
        for tp in range(NW):
            acc = acc + ps_v[pl.ds(tp * RPT + j * L, L)]
        tot_v[pl.ds(j * L, L)] = acc

    @plsc.parallel_loop(0, RPT * 4 // L)
    def _(j):
        acc = zv
        for tp in range(NW):
            acc = acc + pc_v[pl.ds(tp * RPT * 4 + j * L, L)]
        totc_v[pl.ds(j * L, L)] = acc

    lanes = lax.iota(I32, L)
    u = use_v[pl.ds(0, L)]

    @plsc.parallel_loop(0, RPT // L, unroll=2)
    def _(j):
        sl = pl.ds(j * L, L)
        bg = jnp.exp(-tot_v[sl])
        gi = (lanes + j * L) * 4
        wr = plsc.load_gather(totc_v, [gi])
        wg = plsc.load_gather(totc_v, [gi + 1])
        wb = plsc.load_gather(totc_v, [gi + 2])
        wz = plsc.load_gather(totc_v, [gi + 3])
        dep = jnp.where(u != 0, wz + bg * tex_v[sl], wz)
        bgb_v[sl] = bg
        depb_v[sl] = dep
        oi = (lanes + j * L) * 3
        plsc.store_scatter(rgbb_v, [oi], wr)
        plsc.store_scatter(rgbb_v, [oi + 1], wg)
        plsc.store_scatter(rgbb_v, [oi + 2], wb)

    pltpu.sync_copy(rgbb_v, rgb_o.at[pl.ds(rbase * 3, RPT * 3)])
    pltpu.sync_copy(depb_v, dep_o.at[pl.ds(rbase, RPT)])
    pltpu.sync_copy(bgb_v, bg_o.at[pl.ds(rbase, RPT)])


def kernel(rgb_samples, radiance_samples, ray_samples_z, ray_samples_dt,
           ray_t_exit, segment_ids, use_ray_t_exit):
    sigma = radiance_samples.reshape(TOTAL)
    rgbf = rgb_samples.T.reshape(TOTAL * 3)
    part = _k1(sigma, ray_samples_dt, segment_ids)
    partf = part.reshape(NW * N_RAYS)
    wout, part4 = _k2(sigma, ray_samples_dt, segment_ids, ray_samples_z,
                      rgbf, partf)
    use_arr = jnp.full((L,), use_ray_t_exit, I32)
    rgbo, dep, bg = _k3(part, part4, ray_t_exit.reshape(N_RAYS), use_arr)
    return (rgbo.reshape(N_RAYS, 3), dep.reshape(N_RAYS, 1),
            bg.reshape(N_RAYS, 1), wout.reshape(TOTAL, 1))
